# Initial kernel scaffold; baseline (speedup 1.0000x reference)
import jax
import jax.numpy as jnp
from jax.experimental import pallas as pl


def kernel(user_embed, book_embed, user_ids, pos_book_ids, neg_book_ids):
    raise NotImplementedError("write your pallas kernel here")


# speedup vs baseline: 2.6218x; 2.6218x over previous
import functools

import jax
import jax.numpy as jnp
from jax import lax
from jax.experimental import pallas as pl
from jax.experimental.pallas import tpu as pltpu
from jax.experimental.pallas import tpu_sc as plsc

D = 64
K = 5
PHASES = K + 1
NC = 2
NS = 16
NW = NC * NS
CHUNK = 128
LANES = 16


@functools.lru_cache(maxsize=None)
def _make_scores_kernel(B: int):
    BW = B // NW
    NCH = BW // CHUNK
    mesh = plsc.VectorSubcoreMesh(core_axis_name="c", subcore_axis_name="s")

    @functools.partial(
        pl.kernel,
        out_type=jax.ShapeDtypeStruct((8, B), jnp.float32),
        mesh=mesh,
        compiler_params=pltpu.CompilerParams(
            needs_layout_passes=False, use_tc_tiling_on_sc=False),
        scratch_types=[
            pltpu.VMEM((BW,), jnp.int32),
            pltpu.VMEM((BW,), jnp.int32),
            pltpu.VMEM((K * BW,), jnp.int32),
            pltpu.VMEM((2, PHASES + 1, CHUNK, D), jnp.float32),
            pltpu.VMEM((8, BW), jnp.float32),
            pltpu.SemaphoreType.DMA,
            pltpu.SemaphoreType.DMA,
        ],
    )
    def scores_kernel(user_hbm, book_hbm, uid_hbm, pid_hbm, nid_hbm, out_hbm,
                      uidx_v, pidx_v, nidx_v, rows_v, scores_v, sem0, sem1):
        wid = lax.axis_index("s") * NC + lax.axis_index("c")
        base = wid * BW

        pltpu.sync_copy(uid_hbm.at[pl.ds(base, BW)], uidx_v)
        pltpu.sync_copy(pid_hbm.at[pl.ds(base, BW)], pidx_v)
        for kk in range(K):
            pltpu.sync_copy(nid_hbm.at[pl.ds(kk * B + base, BW)],
                            nidx_v.at[pl.ds(kk * BW, BW)])

        sems = (sem0, sem1)

        def issue(c):
            buf = c % 2
            cb = c * CHUNK
            sem = sems[buf]
            cps = [
                pltpu.async_copy(
                    user_hbm.at[uidx_v.at[pl.ds(cb, CHUNK)]],
                    rows_v.at[buf, 0], sem),
                pltpu.async_copy(
                    book_hbm.at[pidx_v.at[pl.ds(cb, CHUNK)]],
                    rows_v.at[buf, 1], sem),
            ]
            for kk in range(K):
                cps.append(pltpu.async_copy(
                    book_hbm.at[nidx_v.at[pl.ds(kk * BW + cb, CHUNK)]],
                    rows_v.at[buf, 2 + kk], sem))
            return cps

        lane = lax.iota(jnp.int32, LANES)
        mask_last = lane == (LANES - 1)
        phase_idx = [jnp.full((LANES,), p, jnp.int32) for p in range(PHASES)]

        def compute(c):
            buf = c % 2
            cb = c * CHUNK

            @plsc.parallel_loop(0, CHUNK, 1, unroll=2)
            def _(r):
                us = [rows_v[buf, 0, r, pl.ds(16 * i, 16)] for i in range(4)]
                col = jnp.full((LANES,), cb + r, jnp.int32)
                for p in range(PHASES):
                    vs = [rows_v[buf, 1 + p, r, pl.ds(16 * i, 16)]
                          for i in range(4)]
                    t = ((us[0] * vs[0] + us[1] * vs[1])
                         + (us[2] * vs[2] + us[3] * vs[3]))
                    cum = plsc.cumsum(t)
                    plsc.store_scatter(scores_v, [phase_idx[p], col], cum,
                                       mask=mask_last)

        pending = {0: issue(0)}
        for c in range(NCH):
            if c + 1 < NCH:
                pending[c + 1] = issue(c + 1)
            for cp in pending.pop(c):
                cp.wait()
            compute(c)

        pltpu.sync_copy(scores_v, out_hbm.at[:, pl.ds(base, BW)])

    return scores_kernel


@functools.lru_cache(maxsize=None)
def _make_loss_kernel(B: int):
    def loss_body(scores_ref, out_ref):
        x = scores_ref[...]
        pos = x[0:1, :]
        neg = x[1:PHASES, :]
        lp = jnp.log(jax.nn.sigmoid(pos) + 1e-10)
        ln = jnp.log(jax.nn.sigmoid(-neg) + 1e-10)
        out_ref[0, 0] = -(jnp.sum(lp) + jnp.sum(ln)) / B

    return pl.pallas_call(
        loss_body,
        out_shape=jax.ShapeDtypeStruct((1, 1), jnp.float32),
        out_specs=pl.BlockSpec(memory_space=pltpu.SMEM),
    )


def kernel(user_embed, book_embed, user_ids, pos_book_ids, neg_book_ids):
    B = user_ids.shape[0]
    uid = user_ids.astype(jnp.int32)
    pid = pos_book_ids.astype(jnp.int32)
    nidT = neg_book_ids.astype(jnp.int32).T.reshape(-1)
    scores = _make_scores_kernel(B)(user_embed, book_embed, uid, pid, nidT)
    loss = _make_loss_kernel(B)(scores)
    return loss[0, 0]
